# baseline (speedup 1.0000x reference)
import functools
import math

import jax
import jax.numpy as jnp
from jax import lax
from jax.experimental import pallas as pl
from jax.experimental.pallas import tpu as pltpu
from jax.experimental.pallas import tpu_sc as plsc

DIM = 64
LANES = 16
CHUNK = 128
NBUF = 2
SKEW = 65
SCALE = math.sqrt(DIM)


def _scale_to_skew(src, skew, b):
    def rows(i, carry):
        r0 = i * 4
        vals = []
        for dr in range(4):
            for k in range(DIM // LANES):
                vals.append((dr, k, src[b, r0 + dr, pl.ds(k * LANES, LANES)] * SCALE))
        for dr, k, v in vals:
            skew[r0 + dr, pl.ds(k * LANES, LANES)] = v
        return carry

    lax.fori_loop(0, CHUNK // 4, rows, 0)


def _transpose_from_skew(skew, dst, b):
    rows = [jnp.arange(LANES, dtype=jnp.int32) + (q * LANES)
            for q in range(CHUNK // LANES)]

    def per_d(d, carry):
        g = d // 8
        dr = d % 8
        dcol = jnp.full((LANES,), d, dtype=jnp.int32)
        vals = []
        for q in range(CHUNK // LANES):
            vals.append((q, plsc.load_gather(skew, [rows[q], dcol])))
        for q, v in vals:
            dst[b, g, dr, pl.ds(q * LANES, LANES)] = v
        return carry

    lax.fori_loop(0, DIM, per_d, 0)


@functools.lru_cache(maxsize=None)
def _make_gather(NW, NC, n_chunks, batch):
    mesh = plsc.VectorSubcoreMesh(core_axis_name="c", subcore_axis_name="s")
    nblk = batch // CHUNK

    @functools.partial(
        pl.kernel,
        out_type=jax.ShapeDtypeStruct((n_chunks, DIM // 8, nblk, 8, CHUNK),
                                      jnp.float32),
        mesh=mesh,
        compiler_params=pltpu.CompilerParams(
            use_tc_tiling_on_sc=True, needs_layout_passes=False
        ),
        scratch_types=[
            pltpu.VMEM((n_chunks, CHUNK), jnp.int32),
            pltpu.VMEM((NBUF, CHUNK, 2 * DIM), jnp.float32),
            pltpu.VMEM((CHUNK, SKEW), jnp.float32),
            pltpu.VMEM((NBUF, DIM // 8, 8, CHUNK), jnp.float32),
            pltpu.SemaphoreType.DMA,
            pltpu.SemaphoreType.DMA,
            pltpu.SemaphoreType.DMA,
            pltpu.SemaphoreType.DMA,
        ],
    )
    def body(xt_hbm, table_hbm, out_hbm, idx_v, bufg, skew, bufo,
             sg0, sg1, so0, so1):
        semg = (sg0, sg1)
        semo = (so0, so1)
        wid = lax.axis_index("s") * NC + lax.axis_index("c")
        pltpu.sync_copy(xt_hbm.at[:, pl.ds(wid * CHUNK, CHUNK)], idx_v)

        def g_start(j, b):
            pltpu.async_copy(table_hbm.at[idx_v.at[j]], bufg.at[b], semg[b])

        def g_wait(j, b):
            pltpu.make_async_copy(
                table_hbm.at[idx_v.at[j]], bufg.at[b], semg[b]
            ).wait()

        def o_start(j, b):
            pltpu.async_copy(bufo.at[b], out_hbm.at[j, :, wid], semo[b])

        def o_wait(j, b):
            pltpu.make_async_copy(
                bufo.at[b], out_hbm.at[j, :, wid], semo[b]
            ).wait()

        def process(j, b):
            _scale_to_skew(bufg, skew, b)
            _transpose_from_skew(skew, bufo, b)

        for b in range(NBUF):
            g_start(b, b)

        for j in range(NBUF):
            b = j
            g_wait(j, b)
            process(j, b)
            g_start(j + NBUF, b)
            o_start(j, b)

        def outer(i, carry):
            for b in range(NBUF):
                j = i * NBUF + b
                g_wait(j, b)
                o_wait(j - NBUF, b)
                process(j, b)
                g_start(j + NBUF, b)
                o_start(j, b)
            return carry

        lax.fori_loop(1, n_chunks // NBUF - 1, outer, 0)

        for b in range(NBUF):
            j = n_chunks - NBUF + b
            g_wait(j, b)
            o_wait(j - NBUF, b)
            process(j, b)
            o_start(j, b)
        for b in range(NBUF):
            o_wait(n_chunks - NBUF + b, b)

    return body


def kernel(x, table):
    batch, seq = x.shape
    info = plsc.get_sparse_core_info()
    NC, NS = info.num_cores, info.num_subcores
    NW = NC * NS
    xt = jnp.transpose(x.astype(jnp.int32))
    table2 = jnp.pad(table, ((0, 0), (0, DIM)))
    out5 = _make_gather(NW, NC, seq, batch)(xt, table2)
    out = jnp.transpose(out5, (2, 4, 0, 1, 3)).reshape(batch, seq, DIM)
    return out


# speedup vs baseline: 1.5521x; 1.5521x over previous
import functools
import math

import jax
import jax.numpy as jnp
from jax import lax
from jax.experimental import pallas as pl
from jax.experimental.pallas import tpu as pltpu
from jax.experimental.pallas import tpu_sc as plsc

DIM = 64
LANES = 16
CHUNK = 128
GBUF = 4
OBUF = 2
SCALE = math.sqrt(DIM)

ROWS_PER_IT = 4


def _scale_chunk(src, bg, dst, bo):
    def rows(i, carry):
        r0 = i * ROWS_PER_IT
        vals = []
        for dr in range(ROWS_PER_IT):
            for k in range(DIM // LANES):
                sl = pl.ds(k * LANES, LANES)
                vals.append((dr, sl, src[bg, r0 + dr, sl] * SCALE))
        for dr, sl, v in vals:
            dst[bo, r0 + dr, sl] = v
        return carry

    lax.fori_loop(0, CHUNK // ROWS_PER_IT, rows, 0)


@functools.lru_cache(maxsize=None)
def _make_gather(NW, NC, n_chunks, b_per_w, B):
    mesh = plsc.VectorSubcoreMesh(core_axis_name="c", subcore_axis_name="s")

    @functools.partial(
        pl.kernel,
        out_type=jax.ShapeDtypeStruct((B, DIM), jnp.float32),
        mesh=mesh,
        compiler_params=pltpu.CompilerParams(
            use_tc_tiling_on_sc=True, needs_layout_passes=False
        ),
        scratch_types=[
            pltpu.VMEM((n_chunks, CHUNK), jnp.int32),
            pltpu.VMEM((GBUF, CHUNK, 2 * DIM), jnp.float32),
            pltpu.VMEM((OBUF, CHUNK, DIM), jnp.float32),
            pltpu.SemaphoreType.DMA,
            pltpu.SemaphoreType.DMA,
            pltpu.SemaphoreType.DMA,
            pltpu.SemaphoreType.DMA,
            pltpu.SemaphoreType.DMA,
            pltpu.SemaphoreType.DMA,
        ],
    )
    def body(idx_hbm, table_hbm, out_hbm, idx_v, bufg, bufo,
             sg0, sg1, sg2, sg3, so0, so1):
        semg = (sg0, sg1, sg2, sg3)
        semo = (so0, so1)
        wid = lax.axis_index("s") * NC + lax.axis_index("c")
        base = wid * b_per_w
        pltpu.sync_copy(idx_hbm.at[wid], idx_v)

        def g_start(j, b):
            pltpu.async_copy(table_hbm.at[idx_v.at[j]], bufg.at[b], semg[b])

        def g_wait(j, b):
            pltpu.make_async_copy(
                table_hbm.at[idx_v.at[j]], bufg.at[b], semg[b]
            ).wait()

        def o_start(j, b):
            pltpu.async_copy(
                bufo.at[b], out_hbm.at[pl.ds(base + j * CHUNK, CHUNK)], semo[b]
            )

        def o_wait(j, b):
            pltpu.make_async_copy(
                bufo.at[b], out_hbm.at[pl.ds(base + j * CHUNK, CHUNK)], semo[b]
            ).wait()

        for b in range(GBUF):
            g_start(b, b)

        for j in range(GBUF):
            g_wait(j, j % GBUF)
            if j >= OBUF:
                o_wait(j - OBUF, j % OBUF)
            _scale_chunk(bufg, j % GBUF, bufo, j % OBUF)
            g_start(j + GBUF, j % GBUF)
            o_start(j, j % OBUF)

        def outer(i, carry):
            for b in range(GBUF):
                j = i * GBUF + b
                bo = b % OBUF
                g_wait(j, b)
                o_wait(j - OBUF, bo)
                _scale_chunk(bufg, b, bufo, bo)
                g_start(j + GBUF, b)
                o_start(j, bo)
            return carry

        lax.fori_loop(1, n_chunks // GBUF - 1, outer, 0)

        for t in range(GBUF):
            j = n_chunks - GBUF + t
            g_wait(j, j % GBUF)
            o_wait(j - OBUF, (j - OBUF) % OBUF)
            _scale_chunk(bufg, j % GBUF, bufo, j % OBUF)
            o_start(j, j % OBUF)
        for t in range(OBUF):
            j = n_chunks - OBUF + t
            o_wait(j, j % OBUF)

    return body


def kernel(x, table):
    batch, seq = x.shape
    B = batch * seq
    info = plsc.get_sparse_core_info()
    NC, NS = info.num_cores, info.num_subcores
    NW = NC * NS
    b_per_w = B // NW
    n_chunks = b_per_w // CHUNK
    idx = x.reshape(NW, n_chunks, CHUNK).astype(jnp.int32)
    table2 = jnp.pad(table, ((0, 0), (0, DIM)))
    out = _make_gather(NW, NC, n_chunks, b_per_w, B)(idx, table2)
    return out.reshape(batch, seq, DIM)
